# Initial kernel scaffold; baseline (speedup 1.0000x reference)
import jax
import jax.numpy as jnp
from jax.experimental import pallas as pl


def kernel(x, li_wqdown, li_wqup, li_wqr, li_wkdown, li_wkup, li_wkr, li_wh, at_wqdown, at_wqup, at_wqr, at_wkvdown, at_wkup, at_wvup, at_wkr, at_wo, fc1, fc2):
    raise NotImplementedError("write your pallas kernel here")


# speedup vs baseline: 6.7411x; 6.7411x over previous
import functools
import math

import numpy as np
import jax
import jax.numpy as jnp
from jax.experimental import pallas as pl
from jax.experimental.pallas import tpu as pltpu

_EMBD = 768
_NHEAD = 12
_LATENT = 512
_ROPED = 64
_MAXSEQ = 2048
_KK = 64
_HEADD = _EMBD // _NHEAD


def _hadamard(n: int) -> np.ndarray:
    h = np.array([[1.0]], dtype=np.float32)
    while h.shape[0] < n:
        h = np.block([[h, h], [h, -h]])
    return h.astype(np.float32)


def _rot_mat(n: int) -> np.ndarray:
    half = n // 2
    r = np.zeros((n, n), dtype=np.float32)
    for i in range(half):
        r[i, i + half] = 1.0
        r[i + half, i] = -1.0
    return r


def _rope_tables(head_dim: int, max_seq: int):
    freqs = 1.0 / 10000.0 ** (jnp.arange(0, head_dim, 2, dtype=jnp.float32) / head_dim)
    t = jnp.arange(max_seq, dtype=jnp.float32)
    ang = jnp.outer(t, freqs)
    return jnp.tile(jnp.cos(ang), (1, 2)), jnp.tile(jnp.sin(ang), (1, 2))


def _proj_kernel(x_ref, cos_ref, sin_ref, cost_ref, sint_ref,
                 h64_ref, hh_ref, r64_ref, rr_ref,
                 wqd_ref, wqu_ref, wqr_ref, wkd_ref, wku_ref, wkr_ref, wh_ref,
                 aqd_ref, aqu_ref, aqr_ref, akvd_ref, aku_ref, avu_ref, akr_ref,
                 liq_ref, liqr_ref, lik_ref, likr_ref, hw_ref,
                 q2_ref, qr2_ref, k2_ref, v2_ref, kr2_ref):
    f32 = jnp.float32
    dot = functools.partial(jnp.dot, preferred_element_type=f32)
    xb = x_ref[...]
    cos = cos_ref[...]
    sin = sin_ref[...]
    cost = cost_ref[...]
    sint = sint_ref[...]
    h64 = h64_ref[...]
    hh = hh_ref[...]
    r64 = r64_ref[...]
    rr = rr_ref[...]
    inv8 = f32(1.0 / 8.0)

    ql = dot(xb, wqd_ref[...])
    liq_ref[...] = dot(dot(ql, wqu_ref[...]), hh) * inv8
    qr0 = dot(ql, wqr_ref[...])
    qr = qr0 * cost + dot(qr0, rr) * sint
    liqr_ref[...] = dot(qr, hh) * inv8
    ckv = dot(xb, wkd_ref[...])
    lik_ref[...] = dot(dot(ckv, wku_ref[...]), h64) * inv8
    kr0 = dot(xb, wkr_ref[...])
    kr = kr0 * cos + dot(kr0, r64) * sin
    likr_ref[...] = dot(kr, h64) * inv8
    hw_ref[...] = jax.nn.sigmoid(dot(ql, wh_ref[...]))

    nx = xb * jax.lax.rsqrt(jnp.mean(xb * xb, axis=-1, keepdims=True) + 1e-5)
    ql2 = dot(nx, aqd_ref[...])
    q2_ref[...] = dot(ql2, aqu_ref[...])
    qr20 = dot(ql2, aqr_ref[...])
    qr2_ref[...] = qr20 * cost + dot(qr20, rr) * sint
    ckv2 = dot(nx, akvd_ref[...])
    k2_ref[...] = dot(ckv2, aku_ref[...])
    v2_ref[...] = dot(ckv2, avu_ref[...])
    kr20 = dot(nx, akr_ref[...])
    kr2_ref[...] = kr20 * cos + dot(kr20, r64) * sin


def _scores_kernel(liq_ref, liqr_ref, lik_ref, likr_ref, hw_ref, msk_ref,
                   *, bs, seq, nhead, kk):
    f32 = jnp.float32
    i = pl.program_id(0)
    liq = liq_ref[...]
    liqr = liqr_ref[...]
    lik = lik_ref[...]
    likr = likr_ref[...]
    hw = hw_ref[...]
    col = jax.lax.broadcasted_iota(jnp.int32, (bs, seq), 1)
    row = i * bs + jax.lax.broadcasted_iota(jnp.int32, (bs, seq), 0)
    causal = col <= row
    dn = (((1,), (1,)), ((), ()))
    acc = jnp.zeros((bs, seq), f32)
    for h in range(nhead):
        qh = liq[:, h * 64:(h + 1) * 64]
        qrh = liqr[:, h * 64:(h + 1) * 64]
        s = jax.lax.dot_general(qh, lik, dn, preferred_element_type=f32)
        s = s + jax.lax.dot_general(qrh, likr, dn, preferred_element_type=f32)
        s = jnp.where(causal, jnp.maximum(s, f32(0.0)), f32(0.0))
        acc = acc + hw[:, h:h + 1] * s

    def body(_, carry):
        work, msk = carry
        m = jnp.max(work, axis=1, keepdims=True)
        eq = work == m
        idx = jnp.min(jnp.where(eq, col, seq), axis=1, keepdims=True)
        sel = col == idx
        work = jnp.where(sel, f32(-1.0), work)
        msk = jnp.where(sel, f32(1.0), msk)
        return work, msk

    _, msk = jax.lax.fori_loop(0, kk, body, (acc, jnp.zeros((bs, seq), f32)))
    msk_ref[...] = msk


def _attn_kernel(x_ref, q2_ref, qr2_ref, k2_ref, v2_ref, kr2_ref, msk_ref,
                 wo_ref, y_ref, *, nhead, scale):
    f32 = jnp.float32
    q2 = q2_ref[...]
    qr2 = qr2_ref[...]
    k2 = k2_ref[...]
    v2 = v2_ref[...]
    kr2 = kr2_ref[...]
    keep = msk_ref[...] > f32(0.5)
    dn = (((1,), (1,)), ((), ()))
    outs = []
    for h in range(nhead):
        qh = q2[:, h * 64:(h + 1) * 64]
        qrh = qr2[:, h * 64:(h + 1) * 64]
        kh = k2[:, h * 64:(h + 1) * 64]
        vh = v2[:, h * 64:(h + 1) * 64]
        s = jax.lax.dot_general(qh, kh, dn, preferred_element_type=f32)
        s = s + jax.lax.dot_general(qrh, kr2, dn, preferred_element_type=f32)
        s = jnp.where(keep, s * f32(scale), f32(-1e30))
        m = jnp.max(s, axis=1, keepdims=True)
        e = jnp.exp(s - m)
        p = e / jnp.sum(e, axis=1, keepdims=True)
        outs.append(jnp.dot(p, vh, preferred_element_type=f32))
    out = jnp.concatenate(outs, axis=1)
    y_ref[...] = x_ref[...] + jnp.dot(out, wo_ref[...], preferred_element_type=f32)


def _mlp_kernel(y_ref, fc1_ref, fc2_ref, z_ref):
    f32 = jnp.float32
    y = y_ref[...]
    ny = y * jax.lax.rsqrt(jnp.mean(y * y, axis=-1, keepdims=True) + 1e-5)
    h = jnp.maximum(jnp.dot(ny, fc1_ref[...], preferred_element_type=f32), f32(0.0))
    z_ref[...] = y + jnp.dot(h, fc2_ref[...], preferred_element_type=f32)


def _full_spec(shape):
    nd = len(shape)
    return pl.BlockSpec(shape, lambda i, _nd=nd: (0,) * _nd)


def _row_spec(bs, w):
    return pl.BlockSpec((bs, w), lambda i: (i, 0))


def kernel(x, li_wqdown, li_wqup, li_wqr, li_wkdown, li_wkup, li_wkr, li_wh,
           at_wqdown, at_wqup, at_wqr, at_wkvdown, at_wkup, at_wvup, at_wkr,
           at_wo, fc1, fc2):
    f32 = jnp.float32
    b, seq, d = x.shape
    nh, hd, rd, lat = _NHEAD, _HEADD, _ROPED, _LATENT
    kk = min(_KK, seq)
    bs = min(256, seq)
    grid = (seq // bs,)
    x2d = x.reshape(seq, d)

    cos, sin = _rope_tables(rd, _MAXSEQ)
    cos = cos[:seq]
    sin = sin[:seq]
    cost = jnp.tile(cos, (1, nh))
    sint = jnp.tile(sin, (1, nh))
    h64 = jnp.asarray(_hadamard(64))
    hh = jnp.asarray(np.kron(np.eye(nh, dtype=np.float32), _hadamard(64)))
    r64 = jnp.asarray(_rot_mat(64))
    rr = jnp.asarray(np.kron(np.eye(nh, dtype=np.float32), _rot_mat(64)))
    whp = jnp.zeros((lat, 128), f32).at[:, :nh].set(li_wh)

    rows = lambda w: _row_spec(bs, w)
    full = _full_spec

    p1_out = pl.pallas_call(
        _proj_kernel,
        grid=grid,
        in_specs=[rows(d), rows(rd), rows(rd), rows(d), rows(d),
                  full((64, 64)), full((d, d)), full((64, 64)), full((d, d)),
                  full((d, lat)), full((lat, d)), full((lat, d)),
                  full((d, lat)), full((lat, hd)), full((d, rd)), full((lat, 128)),
                  full((d, lat)), full((lat, d)), full((lat, d)),
                  full((d, lat)), full((lat, d)), full((lat, d)), full((d, rd))],
        out_specs=[rows(d), rows(d), rows(hd), rows(rd), rows(128),
                   rows(d), rows(d), rows(d), rows(d), rows(rd)],
        out_shape=[jax.ShapeDtypeStruct((seq, d), f32),
                   jax.ShapeDtypeStruct((seq, d), f32),
                   jax.ShapeDtypeStruct((seq, hd), f32),
                   jax.ShapeDtypeStruct((seq, rd), f32),
                   jax.ShapeDtypeStruct((seq, 128), f32),
                   jax.ShapeDtypeStruct((seq, d), f32),
                   jax.ShapeDtypeStruct((seq, d), f32),
                   jax.ShapeDtypeStruct((seq, d), f32),
                   jax.ShapeDtypeStruct((seq, d), f32),
                   jax.ShapeDtypeStruct((seq, rd), f32)],
    )(x2d, cos, sin, cost, sint, h64, hh, r64, rr,
      li_wqdown, li_wqup, li_wqr, li_wkdown, li_wkup, li_wkr, whp,
      at_wqdown, at_wqup, at_wqr, at_wkvdown, at_wkup, at_wvup, at_wkr)
    liq, liqr, lik, likr, hw, q2, qr2, k2, v2, kr2 = p1_out

    f8 = jnp.float8_e4m3fn
    liq = liq.astype(f8).astype(f32)
    liqr = liqr.astype(f8).astype(f32)
    lik = lik.astype(f8).astype(f32)
    likr = likr.astype(f8).astype(f32)

    msk = pl.pallas_call(
        functools.partial(_scores_kernel, bs=bs, seq=seq, nhead=nh, kk=kk),
        grid=grid,
        in_specs=[rows(d), rows(d), full((seq, hd)), full((seq, rd)), rows(128)],
        out_specs=rows(seq),
        out_shape=jax.ShapeDtypeStruct((seq, seq), f32),
    )(liq, liqr, lik, likr, hw)

    y = pl.pallas_call(
        functools.partial(_attn_kernel, nhead=nh, scale=1.0 / math.sqrt(hd + rd)),
        grid=grid,
        in_specs=[rows(d), rows(d), rows(d), full((seq, d)), full((seq, d)),
                  full((seq, rd)), rows(seq), full((d, d))],
        out_specs=rows(d),
        out_shape=jax.ShapeDtypeStruct((seq, d), f32),
    )(x2d, q2, qr2, k2, v2, kr2, msk, at_wo)

    z = pl.pallas_call(
        _mlp_kernel,
        grid=grid,
        in_specs=[rows(d), full((d, 4 * d)), full((4 * d, d))],
        out_specs=rows(d),
        out_shape=jax.ShapeDtypeStruct((seq, d), f32),
    )(y, fc1, fc2)

    return z.reshape(b, seq, d), jnp.zeros((), f32)


# speedup vs baseline: 16.8608x; 2.5012x over previous
import functools
import math

import numpy as np
import jax
import jax.numpy as jnp
from jax.experimental import pallas as pl
from jax.experimental.pallas import tpu as pltpu

_EMBD = 768
_NHEAD = 12
_LATENT = 512
_ROPED = 64
_MAXSEQ = 2048
_KK = 64
_HEADD = _EMBD // _NHEAD


def _hadamard(n: int) -> np.ndarray:
    h = np.array([[1.0]], dtype=np.float32)
    while h.shape[0] < n:
        h = np.block([[h, h], [h, -h]])
    return h.astype(np.float32)


def _rot_mat(n: int) -> np.ndarray:
    half = n // 2
    r = np.zeros((n, n), dtype=np.float32)
    for i in range(half):
        r[i, i + half] = 1.0
        r[i + half, i] = -1.0
    return r


def _rope_tables(head_dim: int, max_seq: int):
    freqs = 1.0 / 10000.0 ** (jnp.arange(0, head_dim, 2, dtype=jnp.float32) / head_dim)
    t = jnp.arange(max_seq, dtype=jnp.float32)
    ang = jnp.outer(t, freqs)
    return jnp.tile(jnp.cos(ang), (1, 2)), jnp.tile(jnp.sin(ang), (1, 2))


def _proj_kernel(x_ref, cos_ref, sin_ref, cost_ref, sint_ref,
                 h64_ref, hh_ref, r64_ref, rr_ref,
                 wqd_ref, wqu_ref, wqr_ref, wkd_ref, wku_ref, wkr_ref, wh_ref,
                 aqd_ref, aqu_ref, aqr_ref, akvd_ref, aku_ref, avu_ref, akr_ref,
                 liq_ref, liqr_ref, lik_ref, likr_ref, hw_ref,
                 q2_ref, qr2_ref, k2_ref, v2_ref, kr2_ref):
    f32 = jnp.float32
    bf16 = jnp.bfloat16
    dot = functools.partial(jnp.dot, preferred_element_type=f32)
    xb = x_ref[...]
    cos = cos_ref[...]
    sin = sin_ref[...]
    cost = cost_ref[...]
    sint = sint_ref[...]
    h64 = h64_ref[...]
    hh = hh_ref[...]
    r64 = r64_ref[...]
    rr = rr_ref[...]
    inv8 = f32(1.0 / 8.0)

    ql = dot(xb, wqd_ref[...])
    liq_ref[...] = dot(dot(ql, wqu_ref[...]), hh) * inv8
    qr0 = dot(ql, wqr_ref[...])
    qr = qr0 * cost + dot(qr0, rr) * sint
    liqr_ref[...] = dot(qr, hh) * inv8
    ckv = dot(xb, wkd_ref[...])
    lik_ref[...] = dot(dot(ckv, wku_ref[...]), h64) * inv8
    kr0 = dot(xb, wkr_ref[...])
    kr = kr0 * cos + dot(kr0, r64) * sin
    likr_ref[...] = dot(kr, h64) * inv8
    hw_ref[...] = jax.nn.sigmoid(dot(ql, wh_ref[...]))

    nx = xb * jax.lax.rsqrt(jnp.mean(xb * xb, axis=-1, keepdims=True) + 1e-5)
    ql2 = dot(nx, aqd_ref[...])
    q2_ref[...] = dot(ql2, aqu_ref[...]).astype(bf16)
    qr20 = dot(ql2, aqr_ref[...])
    qr2_ref[...] = (qr20 * cost + dot(qr20, rr) * sint).astype(bf16)
    ckv2 = dot(nx, akvd_ref[...])
    k2_ref[...] = dot(ckv2, aku_ref[...]).astype(bf16)
    v2_ref[...] = dot(ckv2, avu_ref[...]).astype(bf16)
    kr20 = dot(nx, akr_ref[...])
    kr2_ref[...] = (kr20 * cos + dot(kr20, r64) * sin).astype(bf16)


def _scores_kernel(liq_ref, liqr_ref, lik_ref, likr_ref, hw_ref, msk_ref,
                   *, bs, seq, nhead, kk):
    f32 = jnp.float32
    i32 = jnp.int32
    i = pl.program_id(0)
    liq = liq_ref[...]
    liqr = liqr_ref[...]
    kcat = jnp.concatenate([lik_ref[...], likr_ref[...]], axis=1)
    hw = hw_ref[...]
    col = jax.lax.broadcasted_iota(i32, (bs, seq), 1)
    row = i * bs + jax.lax.broadcasted_iota(i32, (bs, seq), 0)
    dn = (((1,), (1,)), ((), ()))
    acc = jnp.zeros((bs, seq), f32)
    for h in range(nhead):
        qcat = jnp.concatenate(
            [liq[:, h * 64:(h + 1) * 64], liqr[:, h * 64:(h + 1) * 64]], axis=1)
        s = jax.lax.dot_general(qcat, kcat, dn, preferred_element_type=f32)
        acc = acc + hw[:, h:h + 1] * jnp.maximum(s, f32(0.0))
    acc = jnp.where(col <= row, acc, f32(0.0))

    bits = jax.lax.bitcast_convert_type(acc, i32)
    kkm1 = f32(kk - 1)

    def vbody(_, carry):
        lo, hi = carry
        mid = lo + ((hi - lo) >> 1)
        cnt = jnp.sum(jnp.where(bits > mid, f32(1.0), f32(0.0)),
                      axis=1, keepdims=True)
        pred = cnt <= kkm1
        return jnp.where(pred, lo, mid + 1), jnp.where(pred, mid, hi)

    lo0 = jnp.zeros((bs, 1), i32)
    hi0 = jnp.full((bs, 1), jnp.int32(0x7F7FFFFF))
    _, vkk = jax.lax.fori_loop(0, 31, vbody, (lo0, hi0))

    gtf = jnp.where(bits > vkk, f32(1.0), f32(0.0))
    eqf = jnp.where(bits == vkk, f32(1.0), f32(0.0))
    need = f32(kk) - jnp.sum(gtf, axis=1, keepdims=True)

    def jbody(_, carry):
        lo, hi = carry
        mid = lo + ((hi - lo) >> 1)
        lef = jnp.where(col <= mid, f32(1.0), f32(0.0))
        c = jnp.sum(eqf * lef, axis=1, keepdims=True)
        pred = c >= need
        return jnp.where(pred, lo, mid + 1), jnp.where(pred, mid, hi)

    lo1 = jnp.zeros((bs, 1), i32)
    hi1 = jnp.full((bs, 1), jnp.int32(seq - 1))
    _, jstar = jax.lax.fori_loop(0, 11, jbody, (lo1, hi1))

    lef = jnp.where(col <= jstar, f32(1.0), f32(0.0))
    msk_ref[...] = (gtf + eqf * lef).astype(jnp.bfloat16)


def _attn_kernel(x_ref, q2_ref, qr2_ref, k2_ref, v2_ref, kr2_ref, msk_ref,
                 wo_ref, y_ref, *, nhead, scale):
    f32 = jnp.float32
    bf16 = jnp.bfloat16
    q2 = q2_ref[...]
    qr2 = qr2_ref[...]
    k2 = k2_ref[...]
    v2 = v2_ref[...]
    kr2 = kr2_ref[...]
    keep = msk_ref[...] > jnp.bfloat16(0.5)
    dn = (((1,), (1,)), ((), ()))
    outs = []
    for h in range(nhead):
        qcat = jnp.concatenate(
            [q2[:, h * 64:(h + 1) * 64], qr2[:, h * 64:(h + 1) * 64]], axis=1)
        kcat = jnp.concatenate([k2[:, h * 64:(h + 1) * 64], kr2], axis=1)
        vh = v2[:, h * 64:(h + 1) * 64]
        s = jax.lax.dot_general(qcat, kcat, dn, preferred_element_type=f32)
        s = jnp.where(keep, s * f32(scale), f32(-1e30))
        m = jnp.max(s, axis=1, keepdims=True)
        e = jnp.exp(s - m)
        p = (e / jnp.sum(e, axis=1, keepdims=True)).astype(bf16)
        outs.append(jnp.dot(p, vh, preferred_element_type=f32))
    out = jnp.concatenate(outs, axis=1).astype(bf16)
    y_ref[...] = x_ref[...] + jnp.dot(out, wo_ref[...], preferred_element_type=f32)


def _mlp_kernel(y_ref, fc1_ref, fc2_ref, z_ref):
    f32 = jnp.float32
    bf16 = jnp.bfloat16
    y = y_ref[...]
    ny = (y * jax.lax.rsqrt(jnp.mean(y * y, axis=-1, keepdims=True) + 1e-5))
    h = jnp.maximum(
        jnp.dot(ny.astype(bf16), fc1_ref[...], preferred_element_type=f32),
        f32(0.0)).astype(bf16)
    z_ref[...] = y + jnp.dot(h, fc2_ref[...], preferred_element_type=f32)


def _full_spec(shape):
    nd = len(shape)
    return pl.BlockSpec(shape, lambda i, _nd=nd: (0,) * _nd)


def _row_spec(bs, w):
    return pl.BlockSpec((bs, w), lambda i: (i, 0))


def kernel(x, li_wqdown, li_wqup, li_wqr, li_wkdown, li_wkup, li_wkr, li_wh,
           at_wqdown, at_wqup, at_wqr, at_wkvdown, at_wkup, at_wvup, at_wkr,
           at_wo, fc1, fc2):
    f32 = jnp.float32
    bf16 = jnp.bfloat16
    b, seq, d = x.shape
    nh, hd, rd, lat = _NHEAD, _HEADD, _ROPED, _LATENT
    kk = min(_KK, seq)
    bs = min(256, seq)
    grid = (seq // bs,)
    x2d = x.reshape(seq, d)

    cos, sin = _rope_tables(rd, _MAXSEQ)
    cos = cos[:seq]
    sin = sin[:seq]
    cost = jnp.tile(cos, (1, nh))
    sint = jnp.tile(sin, (1, nh))
    h64 = jnp.asarray(_hadamard(64))
    hh = jnp.asarray(np.kron(np.eye(nh, dtype=np.float32), _hadamard(64)))
    r64 = jnp.asarray(_rot_mat(64))
    rr = jnp.asarray(np.kron(np.eye(nh, dtype=np.float32), _rot_mat(64)))
    whp = jnp.zeros((lat, 128), f32).at[:, :nh].set(li_wh)

    rows = lambda w: _row_spec(bs, w)
    full = _full_spec

    p1_out = pl.pallas_call(
        _proj_kernel,
        grid=grid,
        in_specs=[rows(d), rows(rd), rows(rd), rows(d), rows(d),
                  full((64, 64)), full((d, d)), full((64, 64)), full((d, d)),
                  full((d, lat)), full((lat, d)), full((lat, d)),
                  full((d, lat)), full((lat, hd)), full((d, rd)), full((lat, 128)),
                  full((d, lat)), full((lat, d)), full((lat, d)),
                  full((d, lat)), full((lat, d)), full((lat, d)), full((d, rd))],
        out_specs=[rows(d), rows(d), rows(hd), rows(rd), rows(128),
                   rows(d), rows(d), rows(d), rows(d), rows(rd)],
        out_shape=[jax.ShapeDtypeStruct((seq, d), f32),
                   jax.ShapeDtypeStruct((seq, d), f32),
                   jax.ShapeDtypeStruct((seq, hd), f32),
                   jax.ShapeDtypeStruct((seq, rd), f32),
                   jax.ShapeDtypeStruct((seq, 128), f32),
                   jax.ShapeDtypeStruct((seq, d), bf16),
                   jax.ShapeDtypeStruct((seq, d), bf16),
                   jax.ShapeDtypeStruct((seq, d), bf16),
                   jax.ShapeDtypeStruct((seq, d), bf16),
                   jax.ShapeDtypeStruct((seq, rd), bf16)],
    )(x2d, cos, sin, cost, sint, h64, hh, r64, rr,
      li_wqdown, li_wqup, li_wqr, li_wkdown, li_wkup, li_wkr, whp,
      at_wqdown, at_wqup, at_wqr, at_wkvdown, at_wkup, at_wvup, at_wkr)
    liq, liqr, lik, likr, hw, q2, qr2, k2, v2, kr2 = p1_out

    f8 = jnp.float8_e4m3fn
    liq = liq.astype(f8).astype(bf16)
    liqr = liqr.astype(f8).astype(bf16)
    lik = lik.astype(f8).astype(bf16)
    likr = likr.astype(f8).astype(bf16)

    msk = pl.pallas_call(
        functools.partial(_scores_kernel, bs=bs, seq=seq, nhead=nh, kk=kk),
        grid=grid,
        in_specs=[rows(d), rows(d), full((seq, hd)), full((seq, rd)), rows(128)],
        out_specs=rows(seq),
        out_shape=jax.ShapeDtypeStruct((seq, seq), bf16),
    )(liq, liqr, lik, likr, hw)

    y = pl.pallas_call(
        functools.partial(_attn_kernel, nhead=nh, scale=1.0 / math.sqrt(hd + rd)),
        grid=grid,
        in_specs=[rows(d), rows(d), rows(d), full((seq, d)), full((seq, d)),
                  full((seq, rd)), rows(seq), full((d, d))],
        out_specs=rows(d),
        out_shape=jax.ShapeDtypeStruct((seq, d), f32),
    )(x2d, q2, qr2, k2, v2, kr2, msk, at_wo.astype(bf16))

    z = pl.pallas_call(
        _mlp_kernel,
        grid=grid,
        in_specs=[rows(d), full((d, 4 * d)), full((4 * d, d))],
        out_specs=rows(d),
        out_shape=jax.ShapeDtypeStruct((seq, d), f32),
    )(y, fc1.astype(bf16), fc2.astype(bf16))

    return z.reshape(b, seq, d), jnp.zeros((), f32)


# speedup vs baseline: 17.0740x; 1.0126x over previous
import functools
import math

import numpy as np
import jax
import jax.numpy as jnp
from jax.experimental import pallas as pl
from jax.experimental.pallas import tpu as pltpu

_EMBD = 768
_NHEAD = 12
_LATENT = 512
_ROPED = 64
_MAXSEQ = 2048
_KK = 64
_HEADD = _EMBD // _NHEAD


def _hadamard(n: int) -> np.ndarray:
    h = np.array([[1.0]], dtype=np.float32)
    while h.shape[0] < n:
        h = np.block([[h, h], [h, -h]])
    return h.astype(np.float32)


def _rot_mat(n: int) -> np.ndarray:
    half = n // 2
    r = np.zeros((n, n), dtype=np.float32)
    for i in range(half):
        r[i, i + half] = 1.0
        r[i + half, i] = -1.0
    return r


def _rope_tables(head_dim: int, max_seq: int):
    freqs = 1.0 / 10000.0 ** (jnp.arange(0, head_dim, 2, dtype=jnp.float32) / head_dim)
    t = jnp.arange(max_seq, dtype=jnp.float32)
    ang = jnp.outer(t, freqs)
    return jnp.tile(jnp.cos(ang), (1, 2)), jnp.tile(jnp.sin(ang), (1, 2))


def _proj_kernel(x_ref, cos_ref, sin_ref, cost_ref, sint_ref,
                 h64_ref, hh_ref, r64_ref, rr_ref,
                 wqd_ref, wqu_ref, wqr_ref, wkd_ref, wku_ref, wkr_ref, wh_ref,
                 aqd_ref, aqu_ref, aqr_ref, akvd_ref, aku_ref, avu_ref, akr_ref,
                 liq_ref, liqr_ref, lik_ref, likr_ref, hw_ref,
                 q2_ref, qr2_ref, k2_ref, v2_ref, kr2_ref):
    f32 = jnp.float32
    bf16 = jnp.bfloat16
    dot = functools.partial(jnp.dot, preferred_element_type=f32)
    xb = x_ref[...]
    cos = cos_ref[...]
    sin = sin_ref[...]
    cost = cost_ref[...]
    sint = sint_ref[...]
    h64 = h64_ref[...]
    hh = hh_ref[...]
    r64 = r64_ref[...]
    rr = rr_ref[...]
    inv8 = f32(1.0 / 8.0)

    ql = dot(xb, wqd_ref[...])
    liq_ref[...] = dot(dot(ql, wqu_ref[...]), hh) * inv8
    qr0 = dot(ql, wqr_ref[...])
    qr = qr0 * cost + dot(qr0, rr) * sint
    liqr_ref[...] = dot(qr, hh) * inv8
    ckv = dot(xb, wkd_ref[...])
    lik_ref[...] = dot(dot(ckv, wku_ref[...]), h64) * inv8
    kr0 = dot(xb, wkr_ref[...])
    kr = kr0 * cos + dot(kr0, r64) * sin
    likr_ref[...] = dot(kr, h64) * inv8
    hw_ref[...] = jax.nn.sigmoid(dot(ql, wh_ref[...]))

    nx = xb * jax.lax.rsqrt(jnp.mean(xb * xb, axis=-1, keepdims=True) + 1e-5)
    ql2 = dot(nx, aqd_ref[...])
    q2_ref[...] = dot(ql2, aqu_ref[...]).astype(bf16)
    qr20 = dot(ql2, aqr_ref[...])
    qr2_ref[...] = (qr20 * cost + dot(qr20, rr) * sint).astype(bf16)
    ckv2 = dot(nx, akvd_ref[...])
    k2_ref[...] = dot(ckv2, aku_ref[...]).astype(bf16)
    v2_ref[...] = dot(ckv2, avu_ref[...]).astype(bf16)
    kr20 = dot(nx, akr_ref[...])
    kr2_ref[...] = (kr20 * cos + dot(kr20, r64) * sin).astype(bf16)


def _fused_kernel(x_ref, liq_ref, liqr_ref, lik_ref, likr_ref, hw_ref,
                  q2_ref, qr2_ref, k2_ref, v2_ref, kr2_ref,
                  wo_ref, fc1_ref, fc2_ref, z_ref,
                  *, bs, seq, nhead, kk, scale):
    f32 = jnp.float32
    i32 = jnp.int32
    i = pl.program_id(0)
    liq = liq_ref[...]
    liqr = liqr_ref[...]
    kcat = jnp.concatenate([lik_ref[...], likr_ref[...]], axis=1)
    hw = hw_ref[...]
    col = jax.lax.broadcasted_iota(i32, (bs, seq), 1)
    row = i * bs + jax.lax.broadcasted_iota(i32, (bs, seq), 0)
    dn = (((1,), (1,)), ((), ()))
    acc = jnp.zeros((bs, seq), f32)
    for h in range(nhead):
        qcat = jnp.concatenate(
            [liq[:, h * 64:(h + 1) * 64], liqr[:, h * 64:(h + 1) * 64]], axis=1)
        s = jax.lax.dot_general(qcat, kcat, dn, preferred_element_type=f32)
        acc = acc + hw[:, h:h + 1] * jnp.maximum(s, f32(0.0))
    acc = jnp.where(col <= row, acc, f32(0.0))

    bits = jax.lax.bitcast_convert_type(acc, i32)
    kkm1 = f32(kk - 1)

    def vbody(_, carry):
        lo, hi = carry
        mid = lo + ((hi - lo) >> 1)
        cnt = jnp.sum(jnp.where(bits > mid, f32(1.0), f32(0.0)),
                      axis=1, keepdims=True)
        pred = cnt <= kkm1
        return jnp.where(pred, lo, mid + 1), jnp.where(pred, mid, hi)

    lo0 = jnp.zeros((bs, 1), i32)
    hi0 = jnp.full((bs, 1), jnp.int32(0x7F7FFFFF))
    _, vkk = jax.lax.fori_loop(0, 31, vbody, (lo0, hi0))

    gtf = jnp.where(bits > vkk, f32(1.0), f32(0.0))
    eqf = jnp.where(bits == vkk, f32(1.0), f32(0.0))
    need = f32(kk) - jnp.sum(gtf, axis=1, keepdims=True)

    def jbody(_, carry):
        lo, hi = carry
        mid = lo + ((hi - lo) >> 1)
        lef = jnp.where(col <= mid, f32(1.0), f32(0.0))
        c = jnp.sum(eqf * lef, axis=1, keepdims=True)
        pred = c >= need
        return jnp.where(pred, lo, mid + 1), jnp.where(pred, mid, hi)

    lo1 = jnp.zeros((bs, 1), i32)
    hi1 = jnp.full((bs, 1), jnp.int32(seq - 1))
    _, jstar = jax.lax.fori_loop(0, 11, jbody, (lo1, hi1))

    lef = jnp.where(col <= jstar, f32(1.0), f32(0.0))
    keep = (gtf + eqf * lef) > f32(0.5)

    bf16 = jnp.bfloat16
    q2 = q2_ref[...]
    qr2 = qr2_ref[...]
    k2 = k2_ref[...]
    v2 = v2_ref[...]
    kr2 = kr2_ref[...]
    outs = []
    for h in range(nhead):
        qcat = jnp.concatenate(
            [q2[:, h * 64:(h + 1) * 64], qr2[:, h * 64:(h + 1) * 64]], axis=1)
        kcat2 = jnp.concatenate([k2[:, h * 64:(h + 1) * 64], kr2], axis=1)
        vh = v2[:, h * 64:(h + 1) * 64]
        s = jax.lax.dot_general(qcat, kcat2, dn, preferred_element_type=f32)
        s = jnp.where(keep, s * f32(scale), f32(-1e30))
        m = jnp.max(s, axis=1, keepdims=True)
        e = jnp.exp(s - m)
        p = (e / jnp.sum(e, axis=1, keepdims=True)).astype(bf16)
        outs.append(jnp.dot(p, vh, preferred_element_type=f32))
    out = jnp.concatenate(outs, axis=1).astype(bf16)
    y = x_ref[...] + jnp.dot(out, wo_ref[...], preferred_element_type=f32)

    ny = (y * jax.lax.rsqrt(jnp.mean(y * y, axis=-1, keepdims=True) + 1e-5))
    hmid = jnp.maximum(
        jnp.dot(ny.astype(bf16), fc1_ref[...], preferred_element_type=f32),
        f32(0.0)).astype(bf16)
    z_ref[...] = y + jnp.dot(hmid, fc2_ref[...], preferred_element_type=f32)


def _full_spec(shape):
    nd = len(shape)
    return pl.BlockSpec(shape, lambda i, _nd=nd: (0,) * _nd)


def _row_spec(bs, w):
    return pl.BlockSpec((bs, w), lambda i: (i, 0))


def kernel(x, li_wqdown, li_wqup, li_wqr, li_wkdown, li_wkup, li_wkr, li_wh,
           at_wqdown, at_wqup, at_wqr, at_wkvdown, at_wkup, at_wvup, at_wkr,
           at_wo, fc1, fc2):
    f32 = jnp.float32
    bf16 = jnp.bfloat16
    b, seq, d = x.shape
    nh, hd, rd, lat = _NHEAD, _HEADD, _ROPED, _LATENT
    kk = min(_KK, seq)
    bs = min(256, seq)
    grid = (seq // bs,)
    x2d = x.reshape(seq, d)

    cos, sin = _rope_tables(rd, _MAXSEQ)
    cos = cos[:seq]
    sin = sin[:seq]
    cost = jnp.tile(cos, (1, nh))
    sint = jnp.tile(sin, (1, nh))
    h64 = jnp.asarray(_hadamard(64))
    hh = jnp.asarray(np.kron(np.eye(nh, dtype=np.float32), _hadamard(64)))
    r64 = jnp.asarray(_rot_mat(64))
    rr = jnp.asarray(np.kron(np.eye(nh, dtype=np.float32), _rot_mat(64)))
    whp = jnp.zeros((lat, 128), f32).at[:, :nh].set(li_wh)

    rows = lambda w: _row_spec(bs, w)
    full = _full_spec

    p1_out = pl.pallas_call(
        _proj_kernel,
        grid=grid,
        in_specs=[rows(d), rows(rd), rows(rd), rows(d), rows(d),
                  full((64, 64)), full((d, d)), full((64, 64)), full((d, d)),
                  full((d, lat)), full((lat, d)), full((lat, d)),
                  full((d, lat)), full((lat, hd)), full((d, rd)), full((lat, 128)),
                  full((d, lat)), full((lat, d)), full((lat, d)),
                  full((d, lat)), full((lat, d)), full((lat, d)), full((d, rd))],
        out_specs=[rows(d), rows(d), rows(hd), rows(rd), rows(128),
                   rows(d), rows(d), rows(d), rows(d), rows(rd)],
        out_shape=[jax.ShapeDtypeStruct((seq, d), f32),
                   jax.ShapeDtypeStruct((seq, d), f32),
                   jax.ShapeDtypeStruct((seq, hd), f32),
                   jax.ShapeDtypeStruct((seq, rd), f32),
                   jax.ShapeDtypeStruct((seq, 128), f32),
                   jax.ShapeDtypeStruct((seq, d), bf16),
                   jax.ShapeDtypeStruct((seq, d), bf16),
                   jax.ShapeDtypeStruct((seq, d), bf16),
                   jax.ShapeDtypeStruct((seq, d), bf16),
                   jax.ShapeDtypeStruct((seq, rd), bf16)],
    )(x2d, cos, sin, cost, sint, h64, hh, r64, rr,
      li_wqdown, li_wqup, li_wqr, li_wkdown, li_wkup, li_wkr, whp,
      at_wqdown, at_wqup, at_wqr, at_wkvdown, at_wkup, at_wvup, at_wkr)
    liq, liqr, lik, likr, hw, q2, qr2, k2, v2, kr2 = p1_out

    f8 = jnp.float8_e4m3fn
    liq = liq.astype(f8).astype(bf16)
    liqr = liqr.astype(f8).astype(bf16)
    lik = lik.astype(f8).astype(bf16)
    likr = likr.astype(f8).astype(bf16)

    z = pl.pallas_call(
        functools.partial(_fused_kernel, bs=bs, seq=seq, nhead=nh, kk=kk,
                          scale=1.0 / math.sqrt(hd + rd)),
        grid=grid,
        in_specs=[rows(d), rows(d), rows(d), full((seq, hd)), full((seq, rd)),
                  rows(128), rows(d), rows(d), full((seq, d)), full((seq, d)),
                  full((seq, rd)), full((d, d)), full((d, 4 * d)),
                  full((4 * d, d))],
        out_specs=rows(d),
        out_shape=jax.ShapeDtypeStruct((seq, d), f32),
    )(x2d, liq, liqr, lik, likr, hw, q2, qr2, k2, v2, kr2,
      at_wo.astype(bf16), fc1.astype(bf16), fc2.astype(bf16))

    return z.reshape(b, seq, d), jnp.zeros((), f32)
